# baseline (speedup 1.0000x reference)
import functools

import jax
import jax.numpy as jnp
from jax import lax
from jax.experimental import pallas as pl
from jax.experimental.pallas import tpu as pltpu
from jax.experimental.pallas import tpu_sc as plsc

N_NODES = 10000
N_EDGES = 320000
D_FEAT = 128
N_HIDDEN = 64
N_CLASSES = 16

NUM_CORES = 2
NUM_SUBCORES = 16
NUM_TILES = NUM_CORES * NUM_SUBCORES
N_PAD = 10240
EPT = N_PAD
CHUNK = 128
NCHUNK = EPT // CHUNK
ROWS_PER_TILE = N_PAD // NUM_SUBCORES
RZ = 128
NB = 5
DEG_W = 16

_MESH = plsc.VectorSubcoreMesh(core_axis_name="c", subcore_axis_name="s")


def _zero_fill(buf, nrows, width):
    z = jnp.zeros((16,), jnp.float32)
    for r in range(nrows):
        for c in range(width // 16):
            buf[r, pl.ds(c * 16, 16)] = z


def _make_push(width):
    @functools.partial(
        pl.kernel,
        out_type=jax.ShapeDtypeStruct((NUM_CORES, N_PAD, width), jnp.float32),
        mesh=_MESH,
        compiler_params=pltpu.CompilerParams(use_tc_tiling_on_sc=False),
        scratch_types=[
            pltpu.VMEM((EPT,), jnp.int32),
            pltpu.VMEM((NCHUNK, CHUNK), jnp.int32),
            pltpu.VMEM((NB, CHUNK, width), jnp.float32),
            pltpu.VMEM((RZ, width), jnp.float32),
            pltpu.VMEM_SHARED((N_PAD, width), jnp.float32),
            pltpu.SemaphoreType.DMA((NB,)),
        ],
    )
    def push(g_hbm, src_hbm, dst_hbm, out_hbm, srcv, dstv, rows, stage, acc, sem):
        cid = lax.axis_index("c")
        sid = lax.axis_index("s")
        wid = cid * NUM_SUBCORES + sid

        pltpu.sync_copy(src_hbm.at[wid], srcv)
        pltpu.sync_copy(dst_hbm.at[wid], dstv)

        _zero_fill(stage, RZ, width)
        row0 = sid * ROWS_PER_TILE
        for i in range(ROWS_PER_TILE // RZ):
            pltpu.sync_copy(stage, acc.at[pl.ds(row0 + i * RZ, RZ)])
        plsc.subcore_barrier()

        def gather_desc(j, b):
            return pltpu.make_async_copy(
                g_hbm.at[srcv.at[pl.ds(j * CHUNK, CHUNK)]], rows.at[b], sem.at[b])

        for b in range(NB - 1):
            gather_desc(b, b).start()

        def outer(g, carry):
            for b in range(NB):
                j = g * NB + b
                jn = j + NB - 1
                nxt = (b + NB - 1) % NB

                @pl.when(jn < NCHUNK)
                def _():
                    gather_desc(jn, nxt).start()

                gather_desc(j, b).wait()
                pltpu.sync_copy(rows.at[b], acc.at[dstv.at[j]], add=True)
            return carry

        lax.fori_loop(0, NCHUNK // NB, outer, 0)
        plsc.subcore_barrier()

        for i in range(ROWS_PER_TILE // RZ):
            sl = pl.ds(row0 + i * RZ, RZ)
            pltpu.sync_copy(acc.at[sl], stage)
            pltpu.sync_copy(stage, out_hbm.at[cid, sl])

    return push


_push64 = _make_push(N_HIDDEN)
_push16 = _make_push(N_CLASSES)


@functools.partial(
    pl.kernel,
    out_type=jax.ShapeDtypeStruct((NUM_CORES, N_PAD, DEG_W), jnp.float32),
    mesh=_MESH,
    compiler_params=pltpu.CompilerParams(use_tc_tiling_on_sc=False),
    scratch_types=[
        pltpu.VMEM((NCHUNK, CHUNK), jnp.int32),
        pltpu.VMEM((CHUNK, DEG_W), jnp.float32),
        pltpu.VMEM((RZ, DEG_W), jnp.float32),
        pltpu.VMEM_SHARED((N_PAD, DEG_W), jnp.float32),
    ],
)
def _deg_kernel(dst_hbm, out_hbm, dstv, ones_rows, stage, acc):
    cid = lax.axis_index("c")
    sid = lax.axis_index("s")
    wid = cid * NUM_SUBCORES + sid

    pltpu.sync_copy(dst_hbm.at[wid], dstv)

    one = jnp.ones((16,), jnp.float32)
    for r in range(CHUNK):
        ones_rows[r, pl.ds(0, 16)] = one

    _zero_fill(stage, RZ, DEG_W)
    row0 = sid * ROWS_PER_TILE
    for i in range(ROWS_PER_TILE // RZ):
        pltpu.sync_copy(stage, acc.at[pl.ds(row0 + i * RZ, RZ)])
    plsc.subcore_barrier()

    def body(j, carry):
        pltpu.sync_copy(ones_rows, acc.at[dstv.at[j]], add=True)
        return carry

    lax.fori_loop(0, NCHUNK, body, 0)
    plsc.subcore_barrier()

    for i in range(ROWS_PER_TILE // RZ):
        sl = pl.ds(row0 + i * RZ, RZ)
        pltpu.sync_copy(acc.at[sl], stage)
        pltpu.sync_copy(stage, out_hbm.at[cid, sl])


def _dinv(p0, p1):
    return lax.rsqrt(p0[:, 0:1] + p1[:, 0:1] + 1.0)


def _tc_a_body(p0, p1, x, w1, g1):
    d = _dinv(p0[...], p1[...])
    g1[...] = d * jnp.dot(x[...], w1[...], preferred_element_type=jnp.float32)


def _tc_a(p0, p1, x, w1):
    return pl.pallas_call(
        _tc_a_body,
        out_shape=jax.ShapeDtypeStruct((N_NODES, N_HIDDEN), jnp.float32),
    )(p0, p1, x, w1)


def _tc_b_body(p0, p1, a0, a1, g1, b1, w2, g2):
    d = _dinv(p0[...], p1[...])
    z1 = jnp.maximum(d * (a0[...] + a1[...] + g1[...]) + b1[...], 0.0)
    g2[...] = d * jnp.dot(z1, w2[...], preferred_element_type=jnp.float32)


def _tc_b(p0, p1, a0, a1, g1, b1, w2):
    return pl.pallas_call(
        _tc_b_body,
        out_shape=jax.ShapeDtypeStruct((N_NODES, N_CLASSES), jnp.float32),
    )(p0, p1, a0, a1, g1, b1, w2)


def _tc_c_body(p0, p1, c0, c1, g2, b2, out):
    d = _dinv(p0[...], p1[...])
    z = d * (c0[...] + c1[...] + g2[...]) + b2[...]
    m = jnp.max(z, axis=1, keepdims=True)
    e = jnp.exp(z - m)
    out[...] = z - m - jnp.log(jnp.sum(e, axis=1, keepdims=True))


def _tc_c(p0, p1, c0, c1, g2, b2):
    return pl.pallas_call(
        _tc_c_body,
        out_shape=jax.ShapeDtypeStruct((N_NODES, N_CLASSES), jnp.float32),
    )(p0, p1, c0, c1, g2, b2)


def kernel(x, edge_index, W1, b1, W2, b2):
    pad_e = EPT - N_EDGES // NUM_TILES
    src = jnp.pad(edge_index[0].reshape(NUM_TILES, N_EDGES // NUM_TILES),
                  ((0, 0), (0, pad_e)), constant_values=0)
    dst = jnp.pad(edge_index[1].reshape(NUM_TILES, N_EDGES // NUM_TILES),
                  ((0, 0), (0, pad_e)),
                  constant_values=N_PAD - 1).reshape(NUM_TILES, NCHUNK, CHUNK)
    b1r = b1.reshape(1, N_HIDDEN)
    b2r = b2.reshape(1, N_CLASSES)

    p = _deg_kernel(dst)
    p0, p1 = p[0, :N_NODES], p[1, :N_NODES]
    g1 = _tc_a(p0, p1, x, W1)
    a = _push64(g1, src, dst)
    g2 = _tc_b(p0, p1, a[0, :N_NODES], a[1, :N_NODES], g1, b1r, W2)
    c = _push16(g2, src, dst)
    return _tc_c(p0, p1, c[0, :N_NODES], c[1, :N_NODES], g2, b2r)


# speedup vs baseline: 2.0368x; 2.0368x over previous
import functools

import jax
import jax.numpy as jnp
from jax import lax
from jax.experimental import pallas as pl
from jax.experimental.pallas import tpu as pltpu
from jax.experimental.pallas import tpu_sc as plsc

N_NODES = 10000
N_EDGES = 320000
D_FEAT = 128
N_HIDDEN = 64
N_CLASSES = 16

NUM_CORES = 2
NUM_SUBCORES = 16
NUM_TILES = NUM_CORES * NUM_SUBCORES
EPT = N_EDGES // NUM_TILES
CHUNK = 80
NCHUNK = EPT // CHUNK
N_PAD = 10240
ROWS_PER_TILE = N_PAD // NUM_SUBCORES
RZ = 128
NB = 5
DEG_W = 16
PK = N_PAD * N_CLASSES // 128

_MESH = plsc.VectorSubcoreMesh(core_axis_name="c", subcore_axis_name="s")


def _zero_fill(buf, nrows, width):
    z = jnp.zeros((16,), jnp.float32)
    for r in range(nrows):
        for c in range(width // 16):
            buf[r, pl.ds(c * 16, 16)] = z


def _make_push(width):
    @functools.partial(
        pl.kernel,
        out_type=jax.ShapeDtypeStruct((NUM_CORES, N_PAD, width), jnp.float32),
        mesh=_MESH,
        compiler_params=pltpu.CompilerParams(use_tc_tiling_on_sc=False),
        scratch_types=[
            pltpu.VMEM((EPT,), jnp.int32),
            pltpu.VMEM((NCHUNK, CHUNK), jnp.int32),
            pltpu.VMEM((NB, CHUNK, width), jnp.float32),
            pltpu.VMEM((RZ, width), jnp.float32),
            pltpu.VMEM_SHARED((N_PAD, width), jnp.float32),
            pltpu.SemaphoreType.DMA((NB,)),
        ],
    )
    def push(g_hbm, src_hbm, dst_hbm, out_hbm, srcv, dstv, rows, stage, acc, sem):
        cid = lax.axis_index("c")
        sid = lax.axis_index("s")
        wid = cid * NUM_SUBCORES + sid

        pltpu.sync_copy(src_hbm.at[wid], srcv)
        pltpu.sync_copy(dst_hbm.at[wid], dstv)

        _zero_fill(stage, RZ, width)
        row0 = sid * ROWS_PER_TILE
        for i in range(ROWS_PER_TILE // RZ):
            pltpu.sync_copy(stage, acc.at[pl.ds(row0 + i * RZ, RZ)])
        plsc.subcore_barrier()

        def gather_desc(j, b):
            return pltpu.make_async_copy(
                g_hbm.at[srcv.at[pl.ds(j * CHUNK, CHUNK)]], rows.at[b], sem.at[b])

        for b in range(NB - 1):
            gather_desc(b, b).start()

        def outer(g, carry):
            for b in range(NB):
                j = g * NB + b
                jn = j + NB - 1
                nxt = (b + NB - 1) % NB

                @pl.when(jn < NCHUNK)
                def _():
                    gather_desc(jn, nxt).start()

                gather_desc(j, b).wait()
                pltpu.sync_copy(rows.at[b], acc.at[dstv.at[j]], add=True)
            return carry

        lax.fori_loop(0, NCHUNK // NB, outer, 0)
        plsc.subcore_barrier()

        for i in range(ROWS_PER_TILE // RZ):
            sl = pl.ds(row0 + i * RZ, RZ)
            pltpu.sync_copy(acc.at[sl], stage)
            pltpu.sync_copy(stage, out_hbm.at[cid, sl])

    return push


_push64 = _make_push(N_HIDDEN)
_push16 = _make_push(N_CLASSES)


@functools.partial(
    pl.kernel,
    out_type=jax.ShapeDtypeStruct((NUM_CORES, N_PAD, DEG_W), jnp.float32),
    mesh=_MESH,
    compiler_params=pltpu.CompilerParams(use_tc_tiling_on_sc=False),
    scratch_types=[
        pltpu.VMEM((NCHUNK, CHUNK), jnp.int32),
        pltpu.VMEM((CHUNK, DEG_W), jnp.float32),
        pltpu.VMEM((RZ, DEG_W), jnp.float32),
        pltpu.VMEM_SHARED((N_PAD, DEG_W), jnp.float32),
    ],
)
def _deg_kernel(dst_hbm, out_hbm, dstv, ones_rows, stage, acc):
    cid = lax.axis_index("c")
    sid = lax.axis_index("s")
    wid = cid * NUM_SUBCORES + sid

    pltpu.sync_copy(dst_hbm.at[wid], dstv)

    one = jnp.ones((16,), jnp.float32)
    for r in range(CHUNK):
        ones_rows[r, pl.ds(0, 16)] = one

    _zero_fill(stage, RZ, DEG_W)
    row0 = sid * ROWS_PER_TILE
    for i in range(ROWS_PER_TILE // RZ):
        pltpu.sync_copy(stage, acc.at[pl.ds(row0 + i * RZ, RZ)])
    plsc.subcore_barrier()

    def body(j, carry):
        pltpu.sync_copy(ones_rows, acc.at[dstv.at[j]], add=True)
        return carry

    lax.fori_loop(0, NCHUNK, body, 0)
    plsc.subcore_barrier()

    for i in range(ROWS_PER_TILE // RZ):
        sl = pl.ds(row0 + i * RZ, RZ)
        pltpu.sync_copy(acc.at[sl], stage)
        pltpu.sync_copy(stage, out_hbm.at[cid, sl])


def _tc_a_body(psum, x, w1, g1):
    d = lax.rsqrt(psum[:, 0:1] + 1.0)
    g1[...] = d * jnp.dot(x[...], w1[...], preferred_element_type=jnp.float32)


def _tc_a(psum, x, w1):
    return pl.pallas_call(
        _tc_a_body,
        out_shape=jax.ShapeDtypeStruct((N_NODES, N_HIDDEN), jnp.float32),
    )(psum, x, w1)


def _tc_b_body(psum, asum, g1, b1, w2, g2):
    d = lax.rsqrt(psum[:, 0:1] + 1.0)
    z1 = jnp.maximum(d * (asum[...] + g1[...]) + b1[...], 0.0)
    g2[...] = d * jnp.dot(z1, w2[...], preferred_element_type=jnp.float32)


def _tc_b(psum, asum, g1, b1, w2):
    return pl.pallas_call(
        _tc_b_body,
        out_shape=jax.ShapeDtypeStruct((N_NODES, N_CLASSES), jnp.float32),
    )(psum, asum, g1, b1, w2)


def _tc_c_body(p_ref, c_ref, g2p, b2t, mblk, out):
    p = p_ref[...]
    d = lax.rsqrt(p[0] + p[1] + 1.0)
    c = c_ref[...]
    z = d * (c[0] + c[1] + g2p[...]) + b2t[...]
    m = jnp.max(z, axis=1, keepdims=True)
    e = jnp.exp(z - m)
    s = jnp.dot(e, mblk[...], preferred_element_type=jnp.float32)
    out[...] = z - m - jnp.log(s)


def _tc_c(p128, c128, g2p, b2t, mblk):
    return pl.pallas_call(
        _tc_c_body,
        out_shape=jax.ShapeDtypeStruct((PK, 128), jnp.float32),
    )(p128, c128, g2p, b2t, mblk)


def kernel(x, edge_index, W1, b1, W2, b2):
    src = edge_index[0].reshape(NUM_TILES, EPT)
    dst = edge_index[1].reshape(NUM_TILES, NCHUNK, CHUNK)
    b1r = b1.reshape(1, N_HIDDEN)
    b2t = jnp.tile(b2, 8).reshape(1, 128)
    mblk = jnp.kron(jnp.eye(8, dtype=jnp.float32),
                    jnp.ones((N_CLASSES, N_CLASSES), jnp.float32))

    p = _deg_kernel(dst)
    psum = p[0, :N_NODES] + p[1, :N_NODES]
    g1 = _tc_a(psum, x, W1)
    a = _push64(g1, src, dst)
    asum = a[0, :N_NODES] + a[1, :N_NODES]
    g2 = _tc_b(psum, asum, g1, b1r, W2)
    g2pad = jnp.pad(g2, ((0, N_PAD - N_NODES), (0, 0)))
    c = _push16(g2pad, src, dst)
    out128 = _tc_c(p.reshape(NUM_CORES, PK, 128),
                   c.reshape(NUM_CORES, PK, 128),
                   g2pad.reshape(PK, 128), b2t, mblk)
    return out128.reshape(N_PAD, N_CLASSES)[:N_NODES]
